# baseline (speedup 1.0000x reference)
import functools

import jax
import jax.numpy as jnp
from jax import lax
from jax.experimental import pallas as pl
from jax.experimental.pallas import tpu as pltpu
from jax.experimental.pallas import tpu_sc as plsc

N = 10000
E = 320000
D = 128
NC = 2
NS = 16
NW = NC * NS

NPAD = 10240
CW = NPAD // NS
E_SCAN = E // NS
E_W = E // NW
G = 128
SEG = 2
SEGSZ = E_W // SEG
NCHUNK = 40
SEGPAD = NCHUNK * G
ROWS_A = 624
ROWS_LAST = N - 15 * ROWS_A

_Z16F = functools.partial(jnp.zeros, (16,), jnp.float32)
_Z16I = functools.partial(jnp.zeros, (16,), jnp.int32)


def _rsqrt_newton(x):
    xi = plsc.bitcast(x, jnp.int32)
    yi = jnp.int32(0x5F3759DF) - lax.shift_right_logical(xi, 1)
    y = plsc.bitcast(yi, jnp.float32)
    for _ in range(3):
        y = y * (1.5 - 0.5 * x * y * y)
    return y


def _norm_body(row_hbm, col_hbm, ew_hbm, dinv_out, norm_out,
               colb, ewb, priv, tmp, acc, disb, dinvb, disfull,
               row3, col3, ew3, norm3, degsh, dis_sh):
    c = lax.axis_index("c")
    s = lax.axis_index("s")

    pltpu.sync_copy(col_hbm.at[pl.ds(s * E_SCAN, E_SCAN)], colb)
    pltpu.sync_copy(ew_hbm.at[pl.ds(s * E_SCAN, E_SCAN)], ewb)

    def _zero(i, _):
        priv[pl.ds(i * 16, 16)] = _Z16F()
        return 0
    lax.fori_loop(0, NPAD // 16, _zero, 0)

    def _scat(i, _):
        cv = colb[pl.ds(i * 16, 16)]
        wv = ewb[pl.ds(i * 16, 16)]
        plsc.addupdate_scatter(priv, [cv], wv)
        return 0
    lax.fori_loop(0, E_SCAN // 16, _scat, 0)

    pltpu.sync_copy(priv, degsh.at[s])
    plsc.subcore_barrier()

    def _zacc(i, _):
        acc[pl.ds(i * 16, 16)] = _Z16F()
        return 0
    lax.fori_loop(0, CW // 16, _zacc, 0)
    for k in range(NS):
        pltpu.sync_copy(degsh.at[k, pl.ds(s * CW, CW)], tmp)

        def _add(i, _):
            acc[pl.ds(i * 16, 16)] = acc[pl.ds(i * 16, 16)] + tmp[pl.ds(i * 16, 16)]
            return 0
        lax.fori_loop(0, CW // 16, _add, 0)

    def _newton(i, _):
        x = acc[pl.ds(i * 16, 16)] + 1.0
        y = _rsqrt_newton(x)
        disb[pl.ds(i * 16, 16)] = y
        dinvb[pl.ds(i * 16, 16)] = y * y
        return 0
    lax.fori_loop(0, CW // 16, _newton, 0)

    pltpu.sync_copy(disb, dis_sh.at[pl.ds(s * CW, CW)])

    @pl.when(c == 0)
    def _():
        pltpu.sync_copy(dinvb, dinv_out.at[pl.ds(s * CW, CW)])

    plsc.subcore_barrier()
    pltpu.sync_copy(dis_sh, disfull)

    wid = c * NS + s
    base = wid * E_W
    pltpu.sync_copy(row_hbm.at[pl.ds(base, E_W)], row3)
    pltpu.sync_copy(col_hbm.at[pl.ds(base, E_W)], col3)
    pltpu.sync_copy(ew_hbm.at[pl.ds(base, E_W)], ew3)

    def _nrm(i, _):
        rv = row3[pl.ds(i * 16, 16)]
        cv = col3[pl.ds(i * 16, 16)]
        ev = ew3[pl.ds(i * 16, 16)]
        dr = plsc.load_gather(disfull, [rv])
        dc = plsc.load_gather(disfull, [cv])
        norm3[pl.ds(i * 16, 16)] = dr * ev * dc
        return 0
    lax.fori_loop(0, E_W // 16, _nrm, 0)
    pltpu.sync_copy(norm3, norm_out.at[pl.ds(base, E_W)])


def _hop_body(h_hbm, row_hbm, col_hbm, norm_hbm, parts_out,
              rowe, cole, norme, rbufA, rbufB, rowgA, rowgB, colgA, colgB,
              zbuf, acc, semA, semB, scsemA, scsemB):
    c = lax.axis_index("c")
    s = lax.axis_index("s")
    wid = c * NS + s
    base = wid * E_W

    def _zz(i, _):
        for q in range(8):
            zbuf[i, pl.ds(q * 16, 16)] = _Z16F()
        return 0
    lax.fori_loop(0, 8, _zz, 0)

    @pl.when(s < 15)
    def _():
        for t in range(ROWS_A // 8):
            pltpu.sync_copy(zbuf, acc.at[pl.ds(s * ROWS_A + t * 8, 8)])

    @pl.when(s == 15)
    def _():
        for t in range(ROWS_LAST // 8):
            pltpu.sync_copy(zbuf, acc.at[pl.ds(15 * ROWS_A + t * 8, 8)])

    for t in range((SEGPAD - SEGSZ) // 16):
        rowe[pl.ds(SEGSZ + t * 16, 16)] = _Z16I()
        cole[pl.ds(SEGSZ + t * 16, 16)] = _Z16I()
        norme[pl.ds(SEGSZ + t * 16, 16)] = _Z16F()
    plsc.subcore_barrier()

    def _scat_wait(rbuf, colg, scsem):
        pltpu.make_async_copy(rbuf, acc.at[colg.at[0]], scsem).wait()

    def _step(j, cur_rowg, cur_rbuf, cur_colg, cur_sem, cur_scsem,
              nxt_rowg, nxt_rbuf, nxt_colg, nxt_sem, nxt_scsem):
        pltpu.make_async_copy(h_hbm.at[cur_rowg.at[0]], cur_rbuf, cur_sem).wait()

        def _scale(i, _):
            nv = norme[pl.ds(j * G + i * 16, 16)]
            for e in range(16):
                sv = lax.broadcast(nv[e], (16,))
                r = i * 16 + e
                for q in range(8):
                    cur_rbuf[r, pl.ds(q * 16, 16)] = (
                        cur_rbuf[r, pl.ds(q * 16, 16)] * sv)
            return 0
        lax.fori_loop(0, G // 16, _scale, 0)

        @pl.when(j >= 1)
        def _():
            _scat_wait(nxt_rbuf, nxt_colg, nxt_scsem)

        @pl.when(j < NCHUNK - 1)
        def _():
            nb = (j + 1) * G
            for k in range(8):
                nxt_rowg[0, pl.ds(k * 16, 16)] = rowe[pl.ds(nb + k * 16, 16)]
            pltpu.async_copy(h_hbm.at[nxt_rowg.at[0]], nxt_rbuf, nxt_sem)

        for k in range(8):
            cur_colg[0, pl.ds(k * 16, 16)] = cole[pl.ds(j * G + k * 16, 16)]
        pltpu.async_copy(cur_rbuf, acc.at[cur_colg.at[0]], cur_scsem, add=True)

    for seg in range(SEG):
        sbase = base + seg * SEGSZ
        pltpu.sync_copy(row_hbm.at[pl.ds(sbase, SEGSZ)], rowe.at[pl.ds(0, SEGSZ)])
        pltpu.sync_copy(col_hbm.at[pl.ds(sbase, SEGSZ)], cole.at[pl.ds(0, SEGSZ)])
        pltpu.sync_copy(norm_hbm.at[pl.ds(sbase, SEGSZ)], norme.at[pl.ds(0, SEGSZ)])

        for k in range(8):
            rowgA[0, pl.ds(k * 16, 16)] = rowe[pl.ds(k * 16, 16)]
        pltpu.async_copy(h_hbm.at[rowgA.at[0]], rbufA, semA)

        def _loop(jj, _):
            _step(2 * jj, rowgA, rbufA, colgA, semA, scsemA,
                  rowgB, rbufB, colgB, semB, scsemB)
            _step(2 * jj + 1, rowgB, rbufB, colgB, semB, scsemB,
                  rowgA, rbufA, colgA, semA, scsemA)
            return 0
        lax.fori_loop(0, NCHUNK // 2, _loop, 0)

        _scat_wait(rbufB, colgB, scsemB)

    plsc.subcore_barrier()

    @pl.when(s < 15)
    def _():
        pltpu.sync_copy(acc.at[pl.ds(s * ROWS_A, ROWS_A)],
                        parts_out.at[c, pl.ds(s * ROWS_A, ROWS_A)])

    @pl.when(s == 15)
    def _():
        pltpu.sync_copy(acc.at[pl.ds(15 * ROWS_A, ROWS_LAST)],
                        parts_out.at[c, pl.ds(15 * ROWS_A, ROWS_LAST)])


def _comb_body(p_ref, h_ref, dinv_ref, o_ref):
    o_ref[...] = p_ref[0] + p_ref[1] + dinv_ref[...] * h_ref[...]


def _mm_body(q_ref, h_ref, dinv_ref, w_ref, b_ref, o_ref):
    hh = q_ref[0] + q_ref[1] + dinv_ref[...] * h_ref[...]
    o_ref[...] = lax.dot_general(
        hh, w_ref[...], (((1,), (1,)), ((), ())),
        preferred_element_type=jnp.float32) + b_ref[...]


def _sc_mesh():
    return plsc.VectorSubcoreMesh(core_axis_name="c", subcore_axis_name="s")


def _make_norm():
    return pl.kernel(
        _norm_body,
        out_type=(jax.ShapeDtypeStruct((NPAD,), jnp.float32),
                  jax.ShapeDtypeStruct((E,), jnp.float32)),
        mesh=_sc_mesh(),
        compiler_params=pltpu.CompilerParams(needs_layout_passes=False),
        scratch_types=(
            pltpu.VMEM((E_SCAN,), jnp.int32),
            pltpu.VMEM((E_SCAN,), jnp.float32),
            pltpu.VMEM((NPAD,), jnp.float32),
            pltpu.VMEM((CW,), jnp.float32),
            pltpu.VMEM((CW,), jnp.float32),
            pltpu.VMEM((CW,), jnp.float32),
            pltpu.VMEM((CW,), jnp.float32),
            pltpu.VMEM((NPAD,), jnp.float32),
            pltpu.VMEM((E_W,), jnp.int32),
            pltpu.VMEM((E_W,), jnp.int32),
            pltpu.VMEM((E_W,), jnp.float32),
            pltpu.VMEM((E_W,), jnp.float32),
            pltpu.VMEM_SHARED((NS, NPAD), jnp.float32),
            pltpu.VMEM_SHARED((NPAD,), jnp.float32),
        ),
    )


def _make_hop():
    return pl.kernel(
        _hop_body,
        out_type=jax.ShapeDtypeStruct((NC, N, D), jnp.float32),
        mesh=_sc_mesh(),
        compiler_params=pltpu.CompilerParams(needs_layout_passes=False),
        scratch_types=(
            pltpu.VMEM((SEGPAD,), jnp.int32),
            pltpu.VMEM((SEGPAD,), jnp.int32),
            pltpu.VMEM((SEGPAD,), jnp.float32),
            pltpu.VMEM((G, D), jnp.float32),
            pltpu.VMEM((G, D), jnp.float32),
            pltpu.VMEM((1, G), jnp.int32),
            pltpu.VMEM((1, G), jnp.int32),
            pltpu.VMEM((1, G), jnp.int32),
            pltpu.VMEM((1, G), jnp.int32),
            pltpu.VMEM((8, D), jnp.float32),
            pltpu.VMEM_SHARED((N, D), jnp.float32),
            pltpu.SemaphoreType.DMA,
            pltpu.SemaphoreType.DMA,
            pltpu.SemaphoreType.DMA,
            pltpu.SemaphoreType.DMA,
        ),
    )


def _comb(parts, h, dinv):
    return pl.pallas_call(
        _comb_body,
        out_shape=jax.ShapeDtypeStruct((N, D), jnp.float32),
        grid=(10,),
        in_specs=[
            pl.BlockSpec((NC, N // 10, D), lambda i: (0, i, 0)),
            pl.BlockSpec((N // 10, D), lambda i: (i, 0)),
            pl.BlockSpec((N // 10, 1), lambda i: (i, 0)),
        ],
        out_specs=pl.BlockSpec((N // 10, D), lambda i: (i, 0)),
    )(parts, h, dinv)


def _mm(parts, h, dinv, Wt, b2):
    return pl.pallas_call(
        _mm_body,
        out_shape=jax.ShapeDtypeStruct((N, D), jnp.float32),
        grid=(10,),
        in_specs=[
            pl.BlockSpec((NC, N // 10, D), lambda i: (0, i, 0)),
            pl.BlockSpec((N // 10, D), lambda i: (i, 0)),
            pl.BlockSpec((N // 10, 1), lambda i: (i, 0)),
            pl.BlockSpec((D, D), lambda i: (0, 0)),
            pl.BlockSpec((1, D), lambda i: (0, 0)),
        ],
        out_specs=pl.BlockSpec((N // 10, D), lambda i: (i, 0)),
    )(parts, h, dinv, Wt, b2)


def kernel(x, edge_index, edge_weight, W, b):
    row = edge_index[0]
    col = edge_index[1]
    dinv_pad, normv = _make_norm()(row, col, edge_weight)
    dinv = dinv_pad[:N].reshape(N, 1)
    p1 = _make_hop()(x, row, col, normv)
    h1 = _comb(p1, x, dinv)
    p2 = _make_hop()(h1, row, col, normv)
    return _mm(p2, h1, dinv, W, b.reshape(1, D))


# speedup vs baseline: 1.0933x; 1.0933x over previous
import functools

import jax
import jax.numpy as jnp
from jax import lax
from jax.experimental import pallas as pl
from jax.experimental.pallas import tpu as pltpu
from jax.experimental.pallas import tpu_sc as plsc

N = 10000
E = 320000
D = 128
NC = 2
NS = 16
NW = NC * NS

NPAD = 10240
CW = NPAD // NS
E_SCAN = E // NS
E_W = E // NW
G = 128
SEG = 2
SEGSZ = E_W // SEG
NCHUNK = 40
SEGPAD = NCHUNK * G
ROWS_A = 624
ROWS_LAST = N - 15 * ROWS_A

HB = 2

_Z16F = functools.partial(jnp.zeros, (16,), jnp.float32)
_Z16I = functools.partial(jnp.zeros, (16,), jnp.int32)


def _rsqrt_newton(x):
    xi = plsc.bitcast(x, jnp.int32)
    yi = jnp.int32(0x5F3759DF) - lax.shift_right_logical(xi, 1)
    y = plsc.bitcast(yi, jnp.float32)
    for _ in range(3):
        y = y * (1.5 - 0.5 * x * y * y)
    return y


def _norm_body(row_hbm, col_hbm, ew_hbm, dinv_out, norm_out,
               colb, ewb, priv, tmp, acc, disb, dinvb, disfull,
               row3, col3, ew3, norm3, degsh, dis_sh):
    c = lax.axis_index("c")
    s = lax.axis_index("s")

    pltpu.sync_copy(col_hbm.at[pl.ds(s * E_SCAN, E_SCAN)], colb)
    pltpu.sync_copy(ew_hbm.at[pl.ds(s * E_SCAN, E_SCAN)], ewb)

    def _zero(i, _):
        priv[pl.ds(i * 16, 16)] = _Z16F()
        return 0
    lax.fori_loop(0, NPAD // 16, _zero, 0)

    def _scat(i, _):
        cv = colb[pl.ds(i * 16, 16)]
        wv = ewb[pl.ds(i * 16, 16)]
        plsc.addupdate_scatter(priv, [cv], wv)
        return 0
    lax.fori_loop(0, E_SCAN // 16, _scat, 0)

    pltpu.sync_copy(priv, degsh.at[s])
    plsc.subcore_barrier()

    def _zacc(i, _):
        acc[pl.ds(i * 16, 16)] = _Z16F()
        return 0
    lax.fori_loop(0, CW // 16, _zacc, 0)
    for k in range(NS):
        pltpu.sync_copy(degsh.at[k, pl.ds(s * CW, CW)], tmp)

        def _add(i, _):
            acc[pl.ds(i * 16, 16)] = acc[pl.ds(i * 16, 16)] + tmp[pl.ds(i * 16, 16)]
            return 0
        lax.fori_loop(0, CW // 16, _add, 0)

    def _newton(i, _):
        x = acc[pl.ds(i * 16, 16)] + 1.0
        y = _rsqrt_newton(x)
        disb[pl.ds(i * 16, 16)] = y
        dinvb[pl.ds(i * 16, 16)] = y * y
        return 0
    lax.fori_loop(0, CW // 16, _newton, 0)

    pltpu.sync_copy(disb, dis_sh.at[pl.ds(s * CW, CW)])

    @pl.when(c == 0)
    def _():
        pltpu.sync_copy(dinvb, dinv_out.at[pl.ds(s * CW, CW)])

    plsc.subcore_barrier()
    pltpu.sync_copy(dis_sh, disfull)

    wid = c * NS + s
    base = wid * E_W
    pltpu.sync_copy(row_hbm.at[pl.ds(base, E_W)], row3)
    pltpu.sync_copy(col_hbm.at[pl.ds(base, E_W)], col3)
    pltpu.sync_copy(ew_hbm.at[pl.ds(base, E_W)], ew3)

    def _nrm(i, _):
        rv = row3[pl.ds(i * 16, 16)]
        cv = col3[pl.ds(i * 16, 16)]
        ev = ew3[pl.ds(i * 16, 16)]
        dr = plsc.load_gather(disfull, [rv])
        dc = plsc.load_gather(disfull, [cv])
        norm3[pl.ds(i * 16, 16)] = dr * ev * dc
        return 0
    lax.fori_loop(0, E_W // 16, _nrm, 0)
    pltpu.sync_copy(norm3, norm_out.at[pl.ds(base, E_W)])


def _hop_body(h_hbm, row_hbm, col_hbm, norm_hbm, parts_out,
              rowe, cole, norme, rbufA, rbufB, rowgA, rowgB, colgA, colgB,
              zbuf, acc, semA, semB, scsemA, scsemB):
    c = lax.axis_index("c")
    s = lax.axis_index("s")
    wid = c * NS + s
    base = wid * E_W

    def _zz(i, _):
        for q in range(8):
            zbuf[i, pl.ds(q * 16, 16)] = _Z16F()
        return 0
    lax.fori_loop(0, 8, _zz, 0)

    @pl.when(s < 15)
    def _():
        for t in range(ROWS_A // 8):
            pltpu.sync_copy(zbuf, acc.at[pl.ds(s * ROWS_A + t * 8, 8)])

    @pl.when(s == 15)
    def _():
        for t in range(ROWS_LAST // 8):
            pltpu.sync_copy(zbuf, acc.at[pl.ds(15 * ROWS_A + t * 8, 8)])

    for t in range((SEGPAD - SEGSZ) // 16):
        rowe[pl.ds(SEGSZ + t * 16, 16)] = _Z16I()
        cole[pl.ds(SEGSZ + t * 16, 16)] = _Z16I()
        norme[pl.ds(SEGSZ + t * 16, 16)] = _Z16F()
    plsc.subcore_barrier()

    def _scat_wait(rbuf, colg, scsem):
        pltpu.make_async_copy(rbuf, acc.at[colg.at[0]], scsem).wait()

    def _gath_issue(rowg, rbuf, sem):
        gh = G // HB
        for p in range(HB):
            pltpu.async_copy(h_hbm.at[rowg.at[0, pl.ds(p * gh, gh)]],
                             rbuf.at[pl.ds(p * gh, gh)], sem)

    def _gath_wait(rowg, rbuf, sem):
        gh = G // HB
        for p in range(HB):
            pltpu.make_async_copy(h_hbm.at[rowg.at[0, pl.ds(p * gh, gh)]],
                                  rbuf.at[pl.ds(p * gh, gh)], sem).wait()

    def _step(j, cur_rowg, cur_rbuf, cur_colg, cur_sem, cur_scsem,
              nxt_rowg, nxt_rbuf, nxt_colg, nxt_sem, nxt_scsem):
        @pl.when(j >= 1)
        def _():
            _scat_wait(nxt_rbuf, nxt_colg, nxt_scsem)

        @pl.when(j < NCHUNK - 1)
        def _():
            nb = (j + 1) * G
            for k in range(8):
                nxt_rowg[0, pl.ds(k * 16, 16)] = rowe[pl.ds(nb + k * 16, 16)]
            _gath_issue(nxt_rowg, nxt_rbuf, nxt_sem)

        _gath_wait(cur_rowg, cur_rbuf, cur_sem)

        def _scale(i, _):
            nv = norme[pl.ds(j * G + i * 16, 16)]
            for e in range(16):
                sv = lax.broadcast(nv[e], (16,))
                r = i * 16 + e
                for q in range(8):
                    cur_rbuf[r, pl.ds(q * 16, 16)] = (
                        cur_rbuf[r, pl.ds(q * 16, 16)] * sv)
            return 0
        lax.fori_loop(0, G // 16, _scale, 0)

        for k in range(8):
            cur_colg[0, pl.ds(k * 16, 16)] = cole[pl.ds(j * G + k * 16, 16)]
        pltpu.async_copy(cur_rbuf, acc.at[cur_colg.at[0]], cur_scsem, add=True)

    for seg in range(SEG):
        sbase = base + seg * SEGSZ
        pltpu.sync_copy(row_hbm.at[pl.ds(sbase, SEGSZ)], rowe.at[pl.ds(0, SEGSZ)])
        pltpu.sync_copy(col_hbm.at[pl.ds(sbase, SEGSZ)], cole.at[pl.ds(0, SEGSZ)])
        pltpu.sync_copy(norm_hbm.at[pl.ds(sbase, SEGSZ)], norme.at[pl.ds(0, SEGSZ)])

        for k in range(8):
            rowgA[0, pl.ds(k * 16, 16)] = rowe[pl.ds(k * 16, 16)]
        pltpu.async_copy(h_hbm.at[rowgA.at[0]], rbufA, semA)

        def _loop(jj, _):
            _step(2 * jj, rowgA, rbufA, colgA, semA, scsemA,
                  rowgB, rbufB, colgB, semB, scsemB)
            _step(2 * jj + 1, rowgB, rbufB, colgB, semB, scsemB,
                  rowgA, rbufA, colgA, semA, scsemA)
            return 0
        lax.fori_loop(0, NCHUNK // 2, _loop, 0)

        _scat_wait(rbufB, colgB, scsemB)

    plsc.subcore_barrier()

    @pl.when(s < 15)
    def _():
        pltpu.sync_copy(acc.at[pl.ds(s * ROWS_A, ROWS_A)],
                        parts_out.at[c, pl.ds(s * ROWS_A, ROWS_A)])

    @pl.when(s == 15)
    def _():
        pltpu.sync_copy(acc.at[pl.ds(15 * ROWS_A, ROWS_LAST)],
                        parts_out.at[c, pl.ds(15 * ROWS_A, ROWS_LAST)])


def _comb_body(p_ref, h_ref, dinv_ref, o_ref):
    o_ref[...] = p_ref[0] + p_ref[1] + dinv_ref[...] * h_ref[...]


def _mm_body(q_ref, h_ref, dinv_ref, w_ref, b_ref, o_ref):
    hh = q_ref[0] + q_ref[1] + dinv_ref[...] * h_ref[...]
    o_ref[...] = lax.dot_general(
        hh, w_ref[...], (((1,), (1,)), ((), ())),
        preferred_element_type=jnp.float32) + b_ref[...]


def _sc_mesh():
    return plsc.VectorSubcoreMesh(core_axis_name="c", subcore_axis_name="s")


def _make_norm():
    return pl.kernel(
        _norm_body,
        out_type=(jax.ShapeDtypeStruct((NPAD,), jnp.float32),
                  jax.ShapeDtypeStruct((E,), jnp.float32)),
        mesh=_sc_mesh(),
        compiler_params=pltpu.CompilerParams(needs_layout_passes=False),
        scratch_types=(
            pltpu.VMEM((E_SCAN,), jnp.int32),
            pltpu.VMEM((E_SCAN,), jnp.float32),
            pltpu.VMEM((NPAD,), jnp.float32),
            pltpu.VMEM((CW,), jnp.float32),
            pltpu.VMEM((CW,), jnp.float32),
            pltpu.VMEM((CW,), jnp.float32),
            pltpu.VMEM((CW,), jnp.float32),
            pltpu.VMEM((NPAD,), jnp.float32),
            pltpu.VMEM((E_W,), jnp.int32),
            pltpu.VMEM((E_W,), jnp.int32),
            pltpu.VMEM((E_W,), jnp.float32),
            pltpu.VMEM((E_W,), jnp.float32),
            pltpu.VMEM_SHARED((NS, NPAD), jnp.float32),
            pltpu.VMEM_SHARED((NPAD,), jnp.float32),
        ),
    )


def _make_hop():
    return pl.kernel(
        _hop_body,
        out_type=jax.ShapeDtypeStruct((NC, N, D), jnp.float32),
        mesh=_sc_mesh(),
        compiler_params=pltpu.CompilerParams(needs_layout_passes=False),
        scratch_types=(
            pltpu.VMEM((SEGPAD,), jnp.int32),
            pltpu.VMEM((SEGPAD,), jnp.int32),
            pltpu.VMEM((SEGPAD,), jnp.float32),
            pltpu.VMEM((G, D), jnp.float32),
            pltpu.VMEM((G, D), jnp.float32),
            pltpu.VMEM((1, G), jnp.int32),
            pltpu.VMEM((1, G), jnp.int32),
            pltpu.VMEM((1, G), jnp.int32),
            pltpu.VMEM((1, G), jnp.int32),
            pltpu.VMEM((8, D), jnp.float32),
            pltpu.VMEM_SHARED((N, D), jnp.float32),
            pltpu.SemaphoreType.DMA,
            pltpu.SemaphoreType.DMA,
            pltpu.SemaphoreType.DMA,
            pltpu.SemaphoreType.DMA,
        ),
    )


def _comb(parts, h, dinv):
    return pl.pallas_call(
        _comb_body,
        out_shape=jax.ShapeDtypeStruct((N, D), jnp.float32),
        grid=(10,),
        in_specs=[
            pl.BlockSpec((NC, N // 10, D), lambda i: (0, i, 0)),
            pl.BlockSpec((N // 10, D), lambda i: (i, 0)),
            pl.BlockSpec((N // 10, 1), lambda i: (i, 0)),
        ],
        out_specs=pl.BlockSpec((N // 10, D), lambda i: (i, 0)),
    )(parts, h, dinv)


def _mm(parts, h, dinv, Wt, b2):
    return pl.pallas_call(
        _mm_body,
        out_shape=jax.ShapeDtypeStruct((N, D), jnp.float32),
        grid=(10,),
        in_specs=[
            pl.BlockSpec((NC, N // 10, D), lambda i: (0, i, 0)),
            pl.BlockSpec((N // 10, D), lambda i: (i, 0)),
            pl.BlockSpec((N // 10, 1), lambda i: (i, 0)),
            pl.BlockSpec((D, D), lambda i: (0, 0)),
            pl.BlockSpec((1, D), lambda i: (0, 0)),
        ],
        out_specs=pl.BlockSpec((N // 10, D), lambda i: (i, 0)),
    )(parts, h, dinv, Wt, b2)


def kernel(x, edge_index, edge_weight, W, b):
    row = edge_index[0]
    col = edge_index[1]
    dinv_pad, normv = _make_norm()(row, col, edge_weight)
    dinv = dinv_pad[:N].reshape(N, 1)
    p1 = _make_hop()(x, row, col, normv)
    h1 = _comb(p1, x, dinv)
    p2 = _make_hop()(h1, row, col, normv)
    return _mm(p2, h1, dinv, W, b.reshape(1, D))
